# Initial kernel scaffold; baseline (speedup 1.0000x reference)
import jax
import jax.numpy as jnp
from jax.experimental import pallas as pl


def kernel(x, edge_index, edge_weight, W1, W2):
    raise NotImplementedError("write your pallas kernel here")


# speedup vs baseline: 3.3017x; 3.3017x over previous
import functools

import jax
import jax.numpy as jnp
from jax import lax
from jax.experimental import pallas as pl
from jax.experimental.pallas import tpu as pltpu
from jax.experimental.pallas import tpu_sc as plsc

NC, NS, L = 2, 16, 16
NW = NC * NS
K = 128


def _make_spmm(n_pad, d, epw):
    nchunks = epw // K
    rows_per_sub = n_pad // NS
    zgrp = rows_per_sub // K

    @functools.partial(
        pl.kernel,
        out_type=jax.ShapeDtypeStruct((NC, n_pad, d), jnp.float32),
        mesh=plsc.VectorSubcoreMesh(core_axis_name="c", subcore_axis_name="s"),
        compiler_params=pltpu.CompilerParams(use_tc_tiling_on_sc=False),
        scratch_types=[
            pltpu.VMEM((K,), jnp.int32),
            pltpu.VMEM((K,), jnp.int32),
            pltpu.VMEM((K,), jnp.float32),
            pltpu.VMEM((K, d), jnp.float32),
            pltpu.VMEM_SHARED((n_pad, d), jnp.float32),
            pltpu.SemaphoreType.DMA,
        ],
    )
    def spmm(h_hbm, src_hbm, dst_hbm, w_hbm, out_hbm, src_v, dst_v, w_v,
             rows_v, acc, sem):
        c = lax.axis_index("c")
        s = lax.axis_index("s")
        wid = c * NS + s
        ncg = d // L

        zero = jnp.zeros((L,), jnp.float32)

        def zrow(r, carry):
            for cg in range(ncg):
                rows_v[r, pl.ds(cg * L, L)] = zero
            return carry

        lax.fori_loop(0, K, zrow, 0)
        r0 = s * rows_per_sub
        for g in range(zgrp):
            pltpu.sync_copy(rows_v, acc.at[pl.ds(r0 + g * K, K), :])
        plsc.subcore_barrier()

        ebase = wid * epw

        def chunk(ci, carry):
            off = ebase + ci * K
            pltpu.sync_copy(src_hbm.at[pl.ds(off, K)], src_v)
            pltpu.sync_copy(dst_hbm.at[pl.ds(off, K)], dst_v)
            pltpu.sync_copy(w_hbm.at[pl.ds(off, K)], w_v)
            pltpu.async_copy(h_hbm.at[src_v], rows_v, sem).wait()

            def grp16(g2, carry2):
                e0 = g2 * L
                w16 = w_v[pl.ds(e0, L)]
                for j in range(L):
                    wj = jnp.full((L,), w16[j], jnp.float32)
                    for cg in range(ncg):
                        sl = pl.ds(cg * L, L)
                        rows_v[e0 + j, sl] = rows_v[e0 + j, sl] * wj
                return carry2

            lax.fori_loop(0, K // L, grp16, 0)
            pltpu.sync_copy(rows_v, acc.at[dst_v], add=True)
            return carry

        lax.fori_loop(0, nchunks, chunk, 0)
        plsc.subcore_barrier()

        for g in range(zgrp):
            sl = pl.ds(r0 + g * K, K)
            pltpu.sync_copy(acc.at[sl, :], out_hbm.at[c, sl, :])

    return spmm


def _matmul(x, w, bm=512):
    n, d = x.shape
    h = w.shape[1]

    def body(x_ref, w_ref, o_ref):
        o_ref[...] = jnp.dot(x_ref[...], w_ref[...],
                             preferred_element_type=jnp.float32)

    return pl.pallas_call(
        body,
        grid=(n // bm,),
        in_specs=[pl.BlockSpec((bm, d), lambda i: (i, 0)),
                  pl.BlockSpec((d, h), lambda i: (0, 0))],
        out_specs=pl.BlockSpec((bm, h), lambda i: (i, 0)),
        out_shape=jax.ShapeDtypeStruct((n, h), jnp.float32),
    )(x, w)


def _relu_combine_matmul(p, w, bm=512):
    _, n, d = p.shape
    h = w.shape[1]

    def body(p_ref, w_ref, o_ref):
        hb = jnp.maximum(p_ref[0] + p_ref[1], 0.0)
        o_ref[...] = jnp.dot(hb, w_ref[...], preferred_element_type=jnp.float32)

    return pl.pallas_call(
        body,
        grid=(n // bm,),
        in_specs=[pl.BlockSpec((2, bm, d), lambda i: (0, i, 0)),
                  pl.BlockSpec((d, h), lambda i: (0, 0))],
        out_specs=pl.BlockSpec((bm, h), lambda i: (i, 0)),
        out_shape=jax.ShapeDtypeStruct((n, h), jnp.float32),
    )(p, w)


def _combine(p, bm=512):
    _, n, d = p.shape

    def body(p_ref, o_ref):
        o_ref[...] = p_ref[0] + p_ref[1]

    return pl.pallas_call(
        body,
        grid=(n // bm,),
        in_specs=[pl.BlockSpec((2, bm, d), lambda i: (0, i, 0))],
        out_specs=pl.BlockSpec((bm, d), lambda i: (i, 0)),
        out_shape=jax.ShapeDtypeStruct((n, d), jnp.float32),
    )(p)


def kernel(x, edge_index, edge_weight, W1, W2):
    n, d = x.shape
    h_dim = W1.shape[1]
    c_dim = W2.shape[1]
    e = edge_index.shape[1]

    n_pad = ((n + NS * K - 1) // (NS * K)) * (NS * K)
    epw = ((e + NW - 1) // NW + K - 1) // K * K
    e_pad = NW * epw

    src = jnp.pad(edge_index[0], (0, e_pad - e))
    dst = jnp.pad(edge_index[1], (0, e_pad - e))
    w = jnp.pad(edge_weight, (0, e_pad - e))
    x_pad = jnp.pad(x, ((0, n_pad - n), (0, 0)))

    spmm1 = _make_spmm(n_pad, h_dim, epw)
    spmm2 = _make_spmm(n_pad, c_dim, epw)

    h = _matmul(x_pad, W1)
    p1 = spmm1(h, src, dst, w)
    h2 = _relu_combine_matmul(p1, W2)
    p2 = spmm2(h2, src, dst, w)
    out = _combine(p2)
    return out[:n]


# speedup vs baseline: 6.7124x; 2.0330x over previous
import functools

import jax
import jax.numpy as jnp
from jax import lax
from jax.experimental import pallas as pl
from jax.experimental.pallas import tpu as pltpu
from jax.experimental.pallas import tpu_sc as plsc

NC, NS, L = 2, 16, 16
NW = NC * NS
K = 80
NQ = 6
UNROLL = 6


def _make_spmm(n_pad, d, epw):
    nchunks = epw // K
    assert nchunks % UNROLL == 0
    nrounds = nchunks // UNROLL
    rows_per_sub = n_pad // NS
    zgrp = rows_per_sub // K

    @functools.partial(
        pl.kernel,
        out_type=jax.ShapeDtypeStruct((NC, n_pad, d), jnp.float32),
        mesh=plsc.VectorSubcoreMesh(core_axis_name="c", subcore_axis_name="s"),
        compiler_params=pltpu.CompilerParams(use_tc_tiling_on_sc=False),
        scratch_types=(
            [pltpu.VMEM((2, K), jnp.int32) for _ in range(NQ)]
            + [pltpu.VMEM((K,), jnp.float32) for _ in range(NQ)]
            + [pltpu.VMEM((K, d), jnp.float32) for _ in range(4)]
            + [pltpu.VMEM_SHARED((n_pad, d), jnp.float32)]
            + [pltpu.SemaphoreType.DMA for _ in range(4 + NQ)]
        ),
    )
    def spmm(h_hbm, eidx_hbm, w_hbm, out_hbm, i0, i1, i2, i3, i4, i5,
             w0, w1, w2, w3, w4, w5,
             gb0, gb1, sb0, sb1, acc,
             gs0, gs1, ss0, ss1, is0, is1, is2, is3, is4, is5):
        c = lax.axis_index("c")
        s = lax.axis_index("s")
        wid = c * NS + s
        ncg = d // L
        idx = (i0, i1, i2, i3, i4, i5)
        wbf = (w0, w1, w2, w3, w4, w5)
        isem = (is0, is1, is2, is3, is4, is5)
        gbufs, sbufs = (gb0, gb1), (sb0, sb1)
        gsems, ssems = (gs0, gs1), (ss0, ss1)

        zero = jnp.zeros((L,), jnp.float32)

        def zrow(r, carry):
            for cg in range(ncg):
                gb0[r, pl.ds(cg * L, L)] = zero
            return carry

        lax.fori_loop(0, K, zrow, 0)
        r0 = s * rows_per_sub
        for g in range(zgrp):
            pltpu.sync_copy(gb0, acc.at[pl.ds(r0 + g * K, K), :])
        plsc.subcore_barrier()

        def idx_load(cj, q):
            pltpu.async_copy(eidx_hbm.at[wid, cj], idx[q], isem[q])
            pltpu.async_copy(w_hbm.at[wid, cj], wbf[q], isem[q])

        def idx_wait(cj, q):
            pltpu.make_async_copy(eidx_hbm.at[wid, cj], idx[q], isem[q]).wait()
            pltpu.make_async_copy(w_hbm.at[wid, cj], wbf[q], isem[q]).wait()

        def gather(cj, q, b):
            pltpu.async_copy(h_hbm.at[idx[q].at[0]], gbufs[b], gsems[b])

        for j in range(4):
            idx_load(j, j)
        idx_wait(0, 0)
        gather(0, 0, 0)
        idx_wait(1, 1)
        gather(1, 1, 1)

        def scale(q, gb, sb):
            def grp16(g2, carry2):
                e0 = g2 * L
                w16 = wbf[q][pl.ds(e0, L)]
                for j in range(L):
                    wj = jnp.full((L,), w16[j], jnp.float32)
                    for cg in range(ncg):
                        sl = pl.ds(cg * L, L)
                        sb[e0 + j, sl] = gb[e0 + j, sl] * wj
                return carry2

            lax.fori_loop(0, K // L, grp16, 0)

        def rnd(g, carry):
            for u in range(UNROLL):
                ci = g * UNROLL + u
                b, q = u % 2, u % NQ
                gb, sb = gbufs[b], sbufs[b]
                pltpu.make_async_copy(h_hbm.at[idx[q].at[0]], gb,
                                      gsems[b]).wait()

                @pl.when(ci >= 2)
                def _wait_prev_scatter():
                    pltpu.make_async_copy(sb, acc.at[idx[q].at[1]],
                                          ssems[b]).wait()

                scale(q, gb, sb)

                @pl.when(ci + 2 < nchunks)
                def _next_gather():
                    q2 = (u + 2) % NQ
                    idx_wait(ci + 2, q2)
                    gather(ci + 2, q2, b)

                pltpu.async_copy(sb, acc.at[idx[q].at[1]], ssems[b], add=True)

                @pl.when(ci + 4 < nchunks)
                def _next_idx():
                    idx_load(ci + 4, (u + 4) % NQ)
            return carry

        lax.fori_loop(0, nrounds, rnd, 0)
        for b in range(2):
            q = (nchunks - 2 + b) % NQ
            pltpu.make_async_copy(sbufs[b], acc.at[idx[q].at[1]],
                                  ssems[b]).wait()
        plsc.subcore_barrier()

        for g in range(zgrp):
            sl = pl.ds(r0 + g * K, K)
            pltpu.sync_copy(acc.at[sl, :], out_hbm.at[c, sl, :])

    return spmm


def _matmul(x, w, bm=512):
    n, d = x.shape
    h = w.shape[1]

    def body(x_ref, w_ref, o_ref):
        o_ref[...] = jnp.dot(x_ref[...], w_ref[...],
                             preferred_element_type=jnp.float32)

    return pl.pallas_call(
        body,
        grid=(n // bm,),
        in_specs=[pl.BlockSpec((bm, d), lambda i: (i, 0)),
                  pl.BlockSpec((d, h), lambda i: (0, 0))],
        out_specs=pl.BlockSpec((bm, h), lambda i: (i, 0)),
        out_shape=jax.ShapeDtypeStruct((n, h), jnp.float32),
    )(x, w)


def _relu_combine_matmul(p, w, bm=512):
    _, n, d = p.shape
    h = w.shape[1]

    def body(p_ref, w_ref, o_ref):
        hb = jnp.maximum(p_ref[0] + p_ref[1], 0.0)
        o_ref[...] = jnp.dot(hb, w_ref[...], preferred_element_type=jnp.float32)

    return pl.pallas_call(
        body,
        grid=(n // bm,),
        in_specs=[pl.BlockSpec((2, bm, d), lambda i: (0, i, 0)),
                  pl.BlockSpec((d, h), lambda i: (0, 0))],
        out_specs=pl.BlockSpec((bm, h), lambda i: (i, 0)),
        out_shape=jax.ShapeDtypeStruct((n, h), jnp.float32),
    )(p, w)


def _combine(p, bm=512):
    _, n, d = p.shape

    def body(p_ref, o_ref):
        o_ref[...] = p_ref[0] + p_ref[1]

    return pl.pallas_call(
        body,
        grid=(n // bm,),
        in_specs=[pl.BlockSpec((2, bm, d), lambda i: (0, i, 0))],
        out_specs=pl.BlockSpec((bm, d), lambda i: (i, 0)),
        out_shape=jax.ShapeDtypeStruct((n, d), jnp.float32),
    )(p)


def kernel(x, edge_index, edge_weight, W1, W2):
    n, d = x.shape
    h_dim = W1.shape[1]
    c_dim = W2.shape[1]
    e = edge_index.shape[1]

    n_pad = ((n + NS * K - 1) // (NS * K)) * (NS * K)
    ek = UNROLL * K
    epw = ((e + NW - 1) // NW + ek - 1) // ek * ek
    e_pad = NW * epw
    nchunks = epw // K

    src = jnp.pad(edge_index[0], (0, e_pad - e)).reshape(NW, nchunks, K)
    dst = jnp.pad(edge_index[1], (0, e_pad - e)).reshape(NW, nchunks, K)
    w = jnp.pad(edge_weight, (0, e_pad - e)).reshape(NW, nchunks, K)
    eidx = jnp.stack([src, dst], axis=2)
    x_pad = jnp.pad(x, ((0, n_pad - n), (0, 0)))

    spmm1 = _make_spmm(n_pad, h_dim, epw)
    spmm2 = _make_spmm(n_pad, c_dim, epw)

    h = _matmul(x_pad, W1)
    p1 = spmm1(h, eidx, w)
    h2 = _relu_combine_matmul(p1, W2)
    p2 = spmm2(h2, eidx, w)
    out = _combine(p2)
    return out[:n]
